# baseline (speedup 1.0000x reference)
import functools

import jax
import jax.numpy as jnp
from jax import lax
from jax.experimental import pallas as pl
from jax.experimental.pallas import tpu as pltpu
from jax.experimental.pallas import tpu_sc as plsc

NUM_HEADS = 16
T = 256
NC, NS, L = 1, 16, 16
NW = NC * NS
RPW = T // NW
GROUPS = RPW * T // L
TAB = 961 * NUM_HEADS

_mesh = plsc.VectorSubcoreMesh(
    core_axis_name="c", subcore_axis_name="s", num_cores=1
)


@functools.partial(
    pl.kernel,
    mesh=_mesh,
    compiler_params=pltpu.CompilerParams(
        needs_layout_passes=False, use_tc_tiling_on_sc=True
    ),
    out_type=jax.ShapeDtypeStruct((1, NUM_HEADS, T, T), jnp.float32),
    scratch_types=[
        pltpu.VMEM((TAB,), jnp.float32),
        pltpu.VMEM((RPW, T), jnp.int32),
        pltpu.VMEM((NUM_HEADS, RPW, T), jnp.float32),
        pltpu.SemaphoreType.DMA,
        pltpu.SemaphoreType.DMA,
        pltpu.SemaphoreType.DMA,
        pltpu.SemaphoreType.DMA,
    ],
)
def _gather_bias(tab_hbm, idx_hbm, out_hbm, tabv, idxv, outv, st, si, so0, so1):
    wid = lax.axis_index("s") * NC + lax.axis_index("c")
    row0 = wid * RPW
    ct = pltpu.async_copy(tab_hbm, tabv, st)
    ci = pltpu.async_copy(idx_hbm.at[pl.ds(row0, RPW)], idxv, si)
    ci.wait()
    ct.wait()

    def half(c0):
        def body(g, c):
            r = g >> 3
            col = c0 + (g & 7) * L
            ivec = idxv[r, pl.ds(col, L)]
            vals = [
                plsc.load_gather(tabv, [ivec + h * 961])
                for h in range(NUM_HEADS)
            ]
            for h in range(NUM_HEADS):
                outv[h, r, pl.ds(col, L)] = vals[h]
            return c

        lax.fori_loop(0, GROUPS // 2, body, 0, unroll=2)

    half(0)
    co0 = pltpu.async_copy(
        outv.at[:, :, pl.ds(0, 128)],
        out_hbm.at[0, :, pl.ds(row0, RPW), pl.ds(0, 128)],
        so0,
    )
    half(128)
    co1 = pltpu.async_copy(
        outv.at[:, :, pl.ds(128, 128)],
        out_hbm.at[0, :, pl.ds(row0, RPW), pl.ds(128, 128)],
        so1,
    )
    co0.wait()
    co1.wait()


def kernel(relative_position_bias_table, relative_position_index):
    tab = relative_position_bias_table.T.reshape(-1)
    return _gather_bias(tab, relative_position_index)


# speedup vs baseline: 1.0585x; 1.0585x over previous
import functools

import jax
import jax.numpy as jnp
from jax import lax
from jax.experimental import pallas as pl
from jax.experimental.pallas import tpu as pltpu
from jax.experimental.pallas import tpu_sc as plsc

NUM_HEADS = 16
T = 256
NC, NS, L = 2, 16, 16
NW = NC * NS
RPW = T // NW
GROUPS = RPW * T // L
TAB = 961 * NUM_HEADS

_mesh = plsc.VectorSubcoreMesh(core_axis_name="c", subcore_axis_name="s")


@functools.partial(
    pl.kernel,
    mesh=_mesh,
    compiler_params=pltpu.CompilerParams(
        needs_layout_passes=False, use_tc_tiling_on_sc=True
    ),
    out_type=jax.ShapeDtypeStruct((1, NUM_HEADS, T, T), jnp.float32),
    scratch_types=[
        pltpu.VMEM((TAB,), jnp.float32),
        pltpu.VMEM((RPW, T), jnp.int32),
        pltpu.VMEM((NUM_HEADS, RPW, T), jnp.float32),
        pltpu.SemaphoreType.DMA,
        pltpu.SemaphoreType.DMA,
        pltpu.SemaphoreType.DMA,
        pltpu.SemaphoreType.DMA,
    ],
)
def _gather_bias(tab_hbm, idx_hbm, out_hbm, tabv, idxv, outv, st, si, so0, so1):
    wid = lax.axis_index("s") * NC + lax.axis_index("c")
    row0 = wid * RPW
    ct = pltpu.async_copy(tab_hbm, tabv, st)
    ci = pltpu.async_copy(idx_hbm.at[pl.ds(row0, RPW)], idxv, si)
    ci.wait()
    ct.wait()

    def half(c0):
        def body(g, c):
            r = g >> 3
            col = c0 + (g & 7) * L
            ivec = idxv[r, pl.ds(col, L)]
            vals = [
                plsc.load_gather(tabv, [ivec + h * 961])
                for h in range(NUM_HEADS)
            ]
            for h in range(NUM_HEADS):
                outv[h, r, pl.ds(col, L)] = vals[h]
            return c

        lax.fori_loop(0, GROUPS // 2, body, 0, unroll=2)

    half(0)
    co0 = pltpu.async_copy(
        outv.at[:, :, pl.ds(0, 128)],
        out_hbm.at[0, :, pl.ds(row0, RPW), pl.ds(0, 128)],
        so0,
    )
    half(128)
    co1 = pltpu.async_copy(
        outv.at[:, :, pl.ds(128, 128)],
        out_hbm.at[0, :, pl.ds(row0, RPW), pl.ds(128, 128)],
        so1,
    )
    co0.wait()
    co1.wait()


def kernel(relative_position_bias_table, relative_position_index):
    tab = relative_position_bias_table.T.reshape(-1)
    return _gather_bias(tab, relative_position_index)
